# baseline (speedup 1.0000x reference)
import functools

import jax
import jax.numpy as jnp
from jax import lax
from jax.experimental import pallas as pl
from jax.experimental.pallas import tpu as pltpu
from jax.experimental.pallas import tpu_sc as plsc

_NUM_EMB = 197
_DIM = 768
_NUM_TILES = 32


def _broadcast_table_sc(W, batch):
    b_per_tile = batch // _NUM_TILES
    mesh = plsc.VectorSubcoreMesh(core_axis_name="c", subcore_axis_name="s")

    @functools.partial(
        pl.kernel,
        out_type=jax.ShapeDtypeStruct((batch, _NUM_EMB, _DIM), W.dtype),
        mesh=mesh,
        scratch_types=[
            pltpu.VMEM_SHARED((_NUM_EMB, _DIM), W.dtype),
            pltpu.SemaphoreType.DMA,
        ],
    )
    def k(w_hbm, out_hbm, w_shared, sem):
        core = lax.axis_index("c")
        sub = lax.axis_index("s")
        tile = core * 16 + sub
        base = tile * b_per_tile

        @pl.when(sub == 0)
        def _():
            pltpu.sync_copy(w_hbm, w_shared)

        plsc.subcore_barrier()

        @pl.loop(0, b_per_tile)
        def _(i):
            pltpu.async_copy(w_shared, out_hbm.at[base + i], sem)

        @pl.loop(0, b_per_tile)
        def _(i):
            pltpu.make_async_copy(w_shared, out_hbm.at[base + i], sem).wait()

    return k(W)


def kernel(x, W):
    return _broadcast_table_sc(W, x.shape[0])


# speedup vs baseline: 1.1208x; 1.1208x over previous
import functools

import jax
import jax.numpy as jnp
from jax import lax
from jax.experimental import pallas as pl
from jax.experimental.pallas import tpu as pltpu
from jax.experimental.pallas import tpu_sc as plsc

_NUM_EMB = 197
_DIM = 768
_SPLIT_ROW = 96
_MAX_ROWS = _NUM_EMB - _SPLIT_ROW
_NUM_SUBCORES = 16
_NUM_TILES = 32
_STREAM_BATCHES = 128


def _broadcast_table_sc(W, batch):
    stream_b = _STREAM_BATCHES
    spmem_b = batch - stream_b
    sb_per_sub = stream_b // _NUM_SUBCORES
    pb_per_tile = spmem_b // _NUM_TILES
    mesh = plsc.VectorSubcoreMesh(core_axis_name="c", subcore_axis_name="s")

    @functools.partial(
        pl.kernel,
        out_type=jax.ShapeDtypeStruct((batch, _NUM_EMB, _DIM), W.dtype),
        mesh=mesh,
        scratch_types=[
            pltpu.VMEM((_MAX_ROWS, _DIM), W.dtype),
            pltpu.VMEM_SHARED((_NUM_EMB, _DIM), W.dtype),
            pltpu.SemaphoreType.DMA,
            pltpu.SemaphoreType.DMA,
        ],
    )
    def k(w_hbm, out_hbm, w_tile, w_shared, sem_s, sem_p):
        core = lax.axis_index("c")
        sub = lax.axis_index("s")
        tile = core * _NUM_SUBCORES + sub

        @pl.when(sub == 0)
        def _():
            pltpu.sync_copy(w_hbm, w_shared)

        def do_half(r0, nrows):
            wt = w_tile.at[pl.ds(0, nrows), :]
            pltpu.sync_copy(w_hbm.at[pl.ds(r0, nrows), :], wt)
            sbase = sub * sb_per_sub

            @pl.loop(0, sb_per_sub)
            def _(i):
                pltpu.async_copy(
                    wt, out_hbm.at[sbase + i, pl.ds(r0, nrows), :], sem_s
                )

        @pl.when(core == 0)
        def _():
            do_half(0, _SPLIT_ROW)

        @pl.when(core == 1)
        def _():
            do_half(_SPLIT_ROW, _MAX_ROWS)

        plsc.subcore_barrier()
        pbase = stream_b + tile * pb_per_tile

        @pl.loop(0, pb_per_tile)
        def _(i):
            pltpu.async_copy(w_shared, out_hbm.at[pbase + i], sem_p)

        def drain_half(r0, nrows):
            wt = w_tile.at[pl.ds(0, nrows), :]
            sbase = sub * sb_per_sub

            @pl.loop(0, sb_per_sub)
            def _(i):
                pltpu.make_async_copy(
                    wt, out_hbm.at[sbase + i, pl.ds(r0, nrows), :], sem_s
                ).wait()

        @pl.when(core == 0)
        def _():
            drain_half(0, _SPLIT_ROW)

        @pl.when(core == 1)
        def _():
            drain_half(_SPLIT_ROW, _MAX_ROWS)

        @pl.loop(0, pb_per_tile)
        def _(i):
            pltpu.make_async_copy(w_shared, out_hbm.at[pbase + i], sem_p).wait()

    return k(W)


def kernel(x, W):
    return _broadcast_table_sc(W, x.shape[0])


# speedup vs baseline: 2.6084x; 2.3272x over previous
import functools

import jax
import jax.numpy as jnp
from jax import lax
from jax.experimental import pallas as pl
from jax.experimental.pallas import tpu as pltpu
from jax.experimental.pallas import tpu_sc as plsc

_NUM_EMB = 197
_DIM = 768
_SPLIT_ROW = 96
_MAX_ROWS = _NUM_EMB - _SPLIT_ROW
_NUM_SUBCORES = 16


def _broadcast_table_sc(W, batch):
    b_per_tile = batch // _NUM_SUBCORES
    mesh = plsc.VectorSubcoreMesh(core_axis_name="c", subcore_axis_name="s")

    @functools.partial(
        pl.kernel,
        out_type=jax.ShapeDtypeStruct((_NUM_EMB, batch, _DIM), W.dtype),
        mesh=mesh,
        scratch_types=[
            pltpu.VMEM((_MAX_ROWS, _DIM), W.dtype),
            pltpu.SemaphoreType.DMA,
        ],
    )
    def k(w_hbm, out_hbm, w_tile, sem):
        core = lax.axis_index("c")
        sub = lax.axis_index("s")
        base = sub * b_per_tile

        def do_half(r0, nrows):
            wt = w_tile.at[pl.ds(0, nrows), :]
            pltpu.sync_copy(w_hbm.at[pl.ds(r0, nrows), :], wt)

            @pl.loop(0, b_per_tile)
            def _(i):
                pltpu.async_copy(
                    wt, out_hbm.at[pl.ds(r0, nrows), base + i, :], sem
                )

            @pl.loop(0, b_per_tile)
            def _(i):
                pltpu.make_async_copy(
                    wt, out_hbm.at[pl.ds(r0, nrows), base + i, :], sem
                ).wait()

        @pl.when(core == 0)
        def _():
            do_half(0, _SPLIT_ROW)

        @pl.when(core == 1)
        def _():
            do_half(_SPLIT_ROW, _MAX_ROWS)

    return k(W)


def kernel(x, W):
    out = _broadcast_table_sc(W, x.shape[0])
    return jnp.transpose(out, (1, 0, 2))
